# baseline (speedup 1.0000x reference)
import functools

import jax
import jax.numpy as jnp
from jax import lax
from jax.experimental import pallas as pl
from jax.experimental.pallas import tpu as pltpu
from jax.experimental.pallas import tpu_sc as plsc

_B = 16384
_NC = 2
_NS = 16
_NW = _NC * _NS
_BPW = _B // _NW
_HB = _BPW // 2


_CB = 4096


def _packed_half(n_rows):
    return ((n_rows // 2 + _CB - 1) // _CB) * _CB


_H_U = _packed_half(1000000)
_H_P = _packed_half(100000)


def _tr_body(xlo_ref, xhi_ref, eye_ref, o_ref):
    del eye_ref
    xt_lo = pltpu.bitcast(xlo_ref[...].T.astype(jnp.bfloat16), jnp.float32)
    xt_hi = pltpu.bitcast(xhi_ref[...].T.astype(jnp.bfloat16), jnp.float32)
    o_ref[...] = jnp.concatenate([xt_lo, xt_hi], axis=1)


def _pack_transposed(tab_t, n_rows):
    nblk = (n_rows // 2 + _CB - 1) // _CB
    half = nblk * _CB
    last = (n_rows + _CB - 1) // _CB - 1
    eye = jnp.eye(64, dtype=jnp.float32)
    return pl.pallas_call(
        _tr_body,
        grid=(nblk,),
        in_specs=[
            pl.BlockSpec((64, _CB), lambda i: (0, i)),
            pl.BlockSpec(
                (64, _CB),
                lambda i, _n=nblk, _l=last: (0, jnp.minimum(i + _n, _l))),
            pl.BlockSpec((64, 64), lambda i: (0, 0)),
        ],
        out_specs=pl.BlockSpec((_CB // 2, 128), lambda i: (i, 0)),
        out_shape=jax.ShapeDtypeStruct((half // 2, 128), jnp.float32),
        compiler_params=pltpu.CompilerParams(
            dimension_semantics=("parallel",),
        ),
    )(tab_t, tab_t, eye)


_GC = 128
_NCH = _BPW // _GC


def _sc_gather_body(t_i, t_t, t_o, idx_v, buf_a, buf_b, sem_a, sem_b):
    wid = lax.axis_index("s") * _NC + lax.axis_index("c")
    base = wid * _BPW
    for j in range(_NCH):
        pltpu.sync_copy(t_i.at[pl.ds(base + _GC * j, _GC)], idx_v.at[j])
    bufs = (buf_a, buf_b)
    sems = (sem_a, sem_b)

    def fire(j):
        return pltpu.async_copy(t_t.at[idx_v.at[j]], bufs[j % 2], sems[j % 2])

    def drain(j, d):
        d.wait()
        pltpu.sync_copy(bufs[j % 2], t_o.at[pl.ds(base + _GC * j, _GC)])

    prev = fire(0)
    for j in range(1, _NCH):
        cur = fire(j)
        drain(j - 1, prev)
        prev = cur
    drain(_NCH - 1, prev)


_sc_gather = functools.partial(
    pl.kernel,
    out_type=jax.ShapeDtypeStruct((_B, 128), jnp.float32),
    mesh=plsc.VectorSubcoreMesh(core_axis_name="c", subcore_axis_name="s"),
    scratch_types=[
        pltpu.VMEM((_NCH, _GC), jnp.int32),
        pltpu.VMEM((_GC, 128), jnp.float32),
        pltpu.VMEM((_GC, 128), jnp.float32),
        pltpu.SemaphoreType.DMA,
        pltpu.SemaphoreType.DMA,
    ],
)(_sc_gather_body)


_BB = 2048


def _extract_row(chunk, idx, half):
    bits = pltpu.bitcast(chunk, jnp.int32)
    h = idx >= half
    sel = jnp.where(h, bits[:, 64:128], bits[:, 0:64])
    q = (idx % 2) == 1
    word = jnp.where(q, sel & jnp.int32(-65536), sel << 16)
    return pltpu.bitcast(word, jnp.float32).astype(jnp.bfloat16)


def _onehot_contrib(idx, tab_t, w1k, n):
    bf = jnp.bfloat16
    oh = (idx == lax.broadcasted_iota(jnp.int32, (1, n), 1)).astype(bf)
    t1 = lax.dot_general(oh, tab_t[...], (((1,), (1,)), ((), ())),
                         preferred_element_type=jnp.float32)
    return jnp.dot(t1.astype(bf), w1k[...],
                   preferred_element_type=jnp.float32)


def _mlp_body(uc, pc, ui, pi, pri, sti, ski, coi, bri,
              prt, stt, skt, cot, brt,
              w1u, w1p, w1pr, w1st, w1sk, w1co, w1br,
              b1, w2, b2, wfc_t, bfc, o):
    f32 = jnp.float32
    bf = jnp.bfloat16
    u = _extract_row(uc[...], ui[...], _H_U)
    p = _extract_row(pc[...], pi[...], _H_P)
    acc = jnp.dot(u, w1u[...], preferred_element_type=f32)
    acc += jnp.dot(p, w1p[...], preferred_element_type=f32)
    acc += _onehot_contrib(pri[...], prt, w1pr, 100)
    acc += _onehot_contrib(sti[...], stt, w1st, 8)
    acc += _onehot_contrib(ski[...], skt, w1sk, 16)
    acc += _onehot_contrib(coi[...], cot, w1co, 32)
    acc += _onehot_contrib(bri[...], brt, w1br, 1000)
    h1 = jnp.maximum(acc + b1[...], 0.0)
    h2 = jnp.maximum(
        jnp.dot(h1.astype(bf), w2[...],
                preferred_element_type=f32) + b2[...], 0.0)
    logit = jnp.sum(h2 * wfc_t[...], axis=1, keepdims=True) + bfc[...]
    o[...] = jax.nn.sigmoid(logit)


def _mlp(uc, pc, idx2, tabs_t, W1, b1, W2, b2, Wfc, bfc):
    bf = jnp.bfloat16
    W1 = W1.astype(bf)
    W2 = W2.astype(bf)
    w1_parts = (W1[0:64], W1[64:128], W1[128:160], W1[160:192],
                W1[192:224], W1[224:256], W1[256:288])
    tabs_t = tuple(t.astype(bf) for t in tabs_t)
    grid = (_B // _BB,)
    chunk_spec = pl.BlockSpec((_BB, 128), lambda i: (i, 0))
    idx_spec = pl.BlockSpec((_BB, 1), lambda i: (i, 0))
    full = lambda shape: pl.BlockSpec(shape, lambda i: (0, 0))
    return pl.pallas_call(
        _mlp_body,
        grid=grid,
        in_specs=[
            chunk_spec, chunk_spec,
            idx_spec, idx_spec, idx_spec, idx_spec, idx_spec, idx_spec,
            idx_spec,
            full((32, 100)), full((32, 8)), full((32, 16)), full((32, 32)),
            full((32, 1000)),
            full((64, 256)), full((64, 256)), full((32, 256)), full((32, 256)),
            full((32, 256)), full((32, 256)), full((32, 256)),
            full((1, 256)), full((256, 128)), full((1, 128)),
            full((1, 128)), full((1, 1)),
        ],
        out_specs=pl.BlockSpec((_BB, 1), lambda i: (i, 0)),
        out_shape=jax.ShapeDtypeStruct((_B, 1), jnp.float32),
        compiler_params=pltpu.CompilerParams(
            dimension_semantics=("arbitrary",),
        ),
    )(uc, pc, *idx2, *tabs_t, *w1_parts,
      b1.reshape(1, 256), W2, b2.reshape(1, 128), Wfc.reshape(1, 128),
      bfc.reshape(1, 1))


def kernel(users, products, prices, skintypes, skintones, concerns, brands,
           user_emb, product_emb, price_emb, skintype_emb, skintone_emb,
           concern_emb, brand_emb, W1, b1, W2, b2, Wfc, bfc):
    users = users.astype(jnp.int32)
    products = products.astype(jnp.int32)
    ut128 = _pack_transposed(user_emb.T, 1000000)
    pt128 = _pack_transposed(product_emb.T, 100000)
    uidx = jnp.where(users < _H_U, users, users - _H_U) >> 1
    pidx = jnp.where(products < _H_P, products, products - _H_P) >> 1
    uc = _sc_gather(uidx, ut128)
    pc = _sc_gather(pidx, pt128)
    idx2 = tuple(x.astype(jnp.int32).reshape(_B, 1) for x in
                 (users, products, prices, skintypes, skintones, concerns,
                  brands))
    tabs_t = (price_emb.T, skintype_emb.T, skintone_emb.T, concern_emb.T,
              brand_emb.T)
    return _mlp(uc, pc, idx2, tabs_t, W1, b1, W2, b2, Wfc, bfc)


# speedup vs baseline: 1.1265x; 1.1265x over previous
import functools

import jax
import jax.numpy as jnp
from jax import lax
from jax.experimental import pallas as pl
from jax.experimental.pallas import tpu as pltpu
from jax.experimental.pallas import tpu_sc as plsc

_B = 16384
_NC = 2
_NS = 16
_NW = _NC * _NS
_BPW = _B // _NW
_HB = _BPW // 2


_CB = 8192


def _packed_half(n_rows):
    return ((n_rows // 2 + _CB - 1) // _CB) * _CB


_H_U = _packed_half(1000000)
_H_P = _packed_half(100000)


def _tr_body(xlo_ref, xhi_ref, eye_ref, o_ref):
    del eye_ref
    xt_lo = pltpu.bitcast(xlo_ref[...].T.astype(jnp.bfloat16), jnp.float32)
    xt_hi = pltpu.bitcast(xhi_ref[...].T.astype(jnp.bfloat16), jnp.float32)
    o_ref[...] = jnp.concatenate([xt_lo, xt_hi], axis=1)


def _pack_transposed(tab_t, n_rows):
    nblk = (n_rows // 2 + _CB - 1) // _CB
    half = nblk * _CB
    last = (n_rows + _CB - 1) // _CB - 1
    eye = jnp.eye(64, dtype=jnp.float32)
    return pl.pallas_call(
        _tr_body,
        grid=(nblk,),
        in_specs=[
            pl.BlockSpec((64, _CB), lambda i: (0, i)),
            pl.BlockSpec(
                (64, _CB),
                lambda i, _n=nblk, _l=last: (0, jnp.minimum(i + _n, _l))),
            pl.BlockSpec((64, 64), lambda i: (0, 0)),
        ],
        out_specs=pl.BlockSpec((_CB // 2, 128), lambda i: (i, 0)),
        out_shape=jax.ShapeDtypeStruct((half // 2, 128), jnp.float32),
        compiler_params=pltpu.CompilerParams(
            dimension_semantics=("parallel",),
        ),
    )(tab_t, tab_t, eye)


_GC = 128
_NCH = _BPW // _GC


def _sc_gather_body(t_i, t_t, t_o, idx_v, buf_a, buf_b, sem_a, sem_b):
    wid = lax.axis_index("s") * _NC + lax.axis_index("c")
    base = wid * _BPW
    for j in range(_NCH):
        pltpu.sync_copy(t_i.at[pl.ds(base + _GC * j, _GC)], idx_v.at[j])
    bufs = (buf_a, buf_b)
    sems = (sem_a, sem_b)

    def fire(j):
        return pltpu.async_copy(t_t.at[idx_v.at[j]], bufs[j % 2], sems[j % 2])

    def drain(j, d):
        d.wait()
        pltpu.sync_copy(bufs[j % 2], t_o.at[pl.ds(base + _GC * j, _GC)])

    prev = fire(0)
    for j in range(1, _NCH):
        cur = fire(j)
        drain(j - 1, prev)
        prev = cur
    drain(_NCH - 1, prev)


_sc_gather = functools.partial(
    pl.kernel,
    out_type=jax.ShapeDtypeStruct((_B, 128), jnp.float32),
    mesh=plsc.VectorSubcoreMesh(core_axis_name="c", subcore_axis_name="s"),
    scratch_types=[
        pltpu.VMEM((_NCH, _GC), jnp.int32),
        pltpu.VMEM((_GC, 128), jnp.float32),
        pltpu.VMEM((_GC, 128), jnp.float32),
        pltpu.SemaphoreType.DMA,
        pltpu.SemaphoreType.DMA,
    ],
)(_sc_gather_body)


_BB = 2048


def _extract_row(chunk, idx, half):
    bits = pltpu.bitcast(chunk, jnp.int32)
    h = idx >= half
    sel = jnp.where(h, bits[:, 64:128], bits[:, 0:64])
    q = (idx % 2) == 1
    word = jnp.where(q, sel & jnp.int32(-65536), sel << 16)
    return pltpu.bitcast(word, jnp.float32).astype(jnp.bfloat16)


def _onehot_contrib(idx, tab_t, w1k, n):
    bf = jnp.bfloat16
    oh = (idx == lax.broadcasted_iota(jnp.int32, (1, n), 1)).astype(bf)
    t1 = lax.dot_general(oh, tab_t[...], (((1,), (1,)), ((), ())),
                         preferred_element_type=jnp.float32)
    return jnp.dot(t1.astype(bf), w1k[...],
                   preferred_element_type=jnp.float32)


def _mlp_body(uc, pc, ui, pi, pri, sti, ski, coi, bri,
              prt, stt, skt, cot, brt,
              w1u, w1p, w1pr, w1st, w1sk, w1co, w1br,
              b1, w2, b2, wfc_t, bfc, o):
    f32 = jnp.float32
    bf = jnp.bfloat16
    u = _extract_row(uc[...], ui[...], _H_U)
    p = _extract_row(pc[...], pi[...], _H_P)
    acc = jnp.dot(u, w1u[...], preferred_element_type=f32)
    acc += jnp.dot(p, w1p[...], preferred_element_type=f32)
    acc += _onehot_contrib(pri[...], prt, w1pr, 100)
    acc += _onehot_contrib(sti[...], stt, w1st, 8)
    acc += _onehot_contrib(ski[...], skt, w1sk, 16)
    acc += _onehot_contrib(coi[...], cot, w1co, 32)
    acc += _onehot_contrib(bri[...], brt, w1br, 1000)
    h1 = jnp.maximum(acc + b1[...], 0.0)
    h2 = jnp.maximum(
        jnp.dot(h1.astype(bf), w2[...],
                preferred_element_type=f32) + b2[...], 0.0)
    logit = jnp.sum(h2 * wfc_t[...], axis=1, keepdims=True) + bfc[...]
    o[...] = jax.nn.sigmoid(logit)


def _mlp(uc, pc, idx2, tabs_t, W1, b1, W2, b2, Wfc, bfc):
    bf = jnp.bfloat16
    W1 = W1.astype(bf)
    W2 = W2.astype(bf)
    w1_parts = (W1[0:64], W1[64:128], W1[128:160], W1[160:192],
                W1[192:224], W1[224:256], W1[256:288])
    tabs_t = tuple(t.astype(bf) for t in tabs_t)
    grid = (_B // _BB,)
    chunk_spec = pl.BlockSpec((_BB, 128), lambda i: (i, 0))
    idx_spec = pl.BlockSpec((_BB, 1), lambda i: (i, 0))
    full = lambda shape: pl.BlockSpec(shape, lambda i: (0, 0))
    return pl.pallas_call(
        _mlp_body,
        grid=grid,
        in_specs=[
            chunk_spec, chunk_spec,
            idx_spec, idx_spec, idx_spec, idx_spec, idx_spec, idx_spec,
            idx_spec,
            full((32, 100)), full((32, 8)), full((32, 16)), full((32, 32)),
            full((32, 1000)),
            full((64, 256)), full((64, 256)), full((32, 256)), full((32, 256)),
            full((32, 256)), full((32, 256)), full((32, 256)),
            full((1, 256)), full((256, 128)), full((1, 128)),
            full((1, 128)), full((1, 1)),
        ],
        out_specs=pl.BlockSpec((_BB, 1), lambda i: (i, 0)),
        out_shape=jax.ShapeDtypeStruct((_B, 1), jnp.float32),
        compiler_params=pltpu.CompilerParams(
            dimension_semantics=("arbitrary",),
        ),
    )(uc, pc, *idx2, *tabs_t, *w1_parts,
      b1.reshape(1, 256), W2, b2.reshape(1, 128), Wfc.reshape(1, 128),
      bfc.reshape(1, 1))


def kernel(users, products, prices, skintypes, skintones, concerns, brands,
           user_emb, product_emb, price_emb, skintype_emb, skintone_emb,
           concern_emb, brand_emb, W1, b1, W2, b2, Wfc, bfc):
    users = users.astype(jnp.int32)
    products = products.astype(jnp.int32)
    ut128 = _pack_transposed(user_emb.T, 1000000)
    pt128 = _pack_transposed(product_emb.T, 100000)
    uidx = jnp.where(users < _H_U, users, users - _H_U) >> 1
    pidx = jnp.where(products < _H_P, products, products - _H_P) >> 1
    uc = _sc_gather(uidx, ut128)
    pc = _sc_gather(pidx, pt128)
    idx2 = tuple(x.astype(jnp.int32).reshape(_B, 1) for x in
                 (users, products, prices, skintypes, skintones, concerns,
                  brands))
    tabs_t = (price_emb.T, skintype_emb.T, skintone_emb.T, concern_emb.T,
              brand_emb.T)
    return _mlp(uc, pc, idx2, tabs_t, W1, b1, W2, b2, Wfc, bfc)


# speedup vs baseline: 1.2020x; 1.0670x over previous
import functools

import jax
import jax.numpy as jnp
from jax import lax
from jax.experimental import pallas as pl
from jax.experimental.pallas import tpu as pltpu
from jax.experimental.pallas import tpu_sc as plsc

_B = 16384
_NC = 2
_NS = 16
_NW = _NC * _NS
_BPW = _B // _NW
_HB = _BPW // 2


_CB = 16384


def _packed_half(n_rows):
    return ((n_rows // 2 + _CB - 1) // _CB) * _CB


_H_U = _packed_half(1000000)
_H_P = _packed_half(100000)


def _tr_body(xlo_ref, xhi_ref, eye_ref, o_ref):
    del eye_ref
    xt_lo = pltpu.bitcast(xlo_ref[...].T.astype(jnp.bfloat16), jnp.float32)
    xt_hi = pltpu.bitcast(xhi_ref[...].T.astype(jnp.bfloat16), jnp.float32)
    o_ref[...] = jnp.concatenate([xt_lo, xt_hi], axis=1)


def _pack_transposed(tab_t, n_rows):
    nblk = (n_rows // 2 + _CB - 1) // _CB
    half = nblk * _CB
    last = (n_rows + _CB - 1) // _CB - 1
    eye = jnp.eye(64, dtype=jnp.float32)
    return pl.pallas_call(
        _tr_body,
        grid=(nblk,),
        in_specs=[
            pl.BlockSpec((64, _CB), lambda i: (0, i)),
            pl.BlockSpec(
                (64, _CB),
                lambda i, _n=nblk, _l=last: (0, jnp.minimum(i + _n, _l))),
            pl.BlockSpec((64, 64), lambda i: (0, 0)),
        ],
        out_specs=pl.BlockSpec((_CB // 2, 128), lambda i: (i, 0)),
        out_shape=jax.ShapeDtypeStruct((half // 2, 128), jnp.float32),
        compiler_params=pltpu.CompilerParams(
            dimension_semantics=("parallel",),
        ),
    )(tab_t, tab_t, eye)


_GC = 128
_NCH = _BPW // _GC


def _sc_gather_body(t_i, t_t, t_o, idx_v, buf_a, buf_b, sem_a, sem_b):
    wid = lax.axis_index("s") * _NC + lax.axis_index("c")
    base = wid * _BPW
    for j in range(_NCH):
        pltpu.sync_copy(t_i.at[pl.ds(base + _GC * j, _GC)], idx_v.at[j])
    bufs = (buf_a, buf_b)
    sems = (sem_a, sem_b)

    def fire(j):
        return pltpu.async_copy(t_t.at[idx_v.at[j]], bufs[j % 2], sems[j % 2])

    def drain(j, d):
        d.wait()
        pltpu.sync_copy(bufs[j % 2], t_o.at[pl.ds(base + _GC * j, _GC)])

    prev = fire(0)
    for j in range(1, _NCH):
        cur = fire(j)
        drain(j - 1, prev)
        prev = cur
    drain(_NCH - 1, prev)


_sc_gather = functools.partial(
    pl.kernel,
    out_type=jax.ShapeDtypeStruct((_B, 128), jnp.float32),
    mesh=plsc.VectorSubcoreMesh(core_axis_name="c", subcore_axis_name="s"),
    scratch_types=[
        pltpu.VMEM((_NCH, _GC), jnp.int32),
        pltpu.VMEM((_GC, 128), jnp.float32),
        pltpu.VMEM((_GC, 128), jnp.float32),
        pltpu.SemaphoreType.DMA,
        pltpu.SemaphoreType.DMA,
    ],
)(_sc_gather_body)


_BB = 2048


def _extract_row(chunk, idx, half):
    bits = pltpu.bitcast(chunk, jnp.int32)
    h = idx >= half
    sel = jnp.where(h, bits[:, 64:128], bits[:, 0:64])
    q = (idx % 2) == 1
    word = jnp.where(q, sel & jnp.int32(-65536), sel << 16)
    return pltpu.bitcast(word, jnp.float32).astype(jnp.bfloat16)


def _onehot_contrib(idx, tab_t, w1k, n):
    bf = jnp.bfloat16
    oh = (idx == lax.broadcasted_iota(jnp.int32, (1, n), 1)).astype(bf)
    t1 = lax.dot_general(oh, tab_t[...], (((1,), (1,)), ((), ())),
                         preferred_element_type=jnp.float32)
    return jnp.dot(t1.astype(bf), w1k[...],
                   preferred_element_type=jnp.float32)


def _mlp_body(uc, pc, ui, pi, pri, sti, ski, coi, bri,
              prt, stt, skt, cot, brt,
              w1u, w1p, w1pr, w1st, w1sk, w1co, w1br,
              b1, w2, b2, wfc_t, bfc, o):
    f32 = jnp.float32
    bf = jnp.bfloat16
    u = _extract_row(uc[...], ui[...], _H_U)
    p = _extract_row(pc[...], pi[...], _H_P)
    acc = jnp.dot(u, w1u[...], preferred_element_type=f32)
    acc += jnp.dot(p, w1p[...], preferred_element_type=f32)
    acc += _onehot_contrib(pri[...], prt, w1pr, 100)
    acc += _onehot_contrib(sti[...], stt, w1st, 8)
    acc += _onehot_contrib(ski[...], skt, w1sk, 16)
    acc += _onehot_contrib(coi[...], cot, w1co, 32)
    acc += _onehot_contrib(bri[...], brt, w1br, 1000)
    h1 = jnp.maximum(acc + b1[...], 0.0)
    h2 = jnp.maximum(
        jnp.dot(h1.astype(bf), w2[...],
                preferred_element_type=f32) + b2[...], 0.0)
    logit = jnp.sum(h2 * wfc_t[...], axis=1, keepdims=True) + bfc[...]
    o[...] = jax.nn.sigmoid(logit)


def _mlp(uc, pc, idx2, tabs_t, W1, b1, W2, b2, Wfc, bfc):
    bf = jnp.bfloat16
    W1 = W1.astype(bf)
    W2 = W2.astype(bf)
    w1_parts = (W1[0:64], W1[64:128], W1[128:160], W1[160:192],
                W1[192:224], W1[224:256], W1[256:288])
    tabs_t = tuple(t.astype(bf) for t in tabs_t)
    grid = (_B // _BB,)
    chunk_spec = pl.BlockSpec((_BB, 128), lambda i: (i, 0))
    idx_spec = pl.BlockSpec((_BB, 1), lambda i: (i, 0))
    full = lambda shape: pl.BlockSpec(shape, lambda i: (0, 0))
    return pl.pallas_call(
        _mlp_body,
        grid=grid,
        in_specs=[
            chunk_spec, chunk_spec,
            idx_spec, idx_spec, idx_spec, idx_spec, idx_spec, idx_spec,
            idx_spec,
            full((32, 100)), full((32, 8)), full((32, 16)), full((32, 32)),
            full((32, 1000)),
            full((64, 256)), full((64, 256)), full((32, 256)), full((32, 256)),
            full((32, 256)), full((32, 256)), full((32, 256)),
            full((1, 256)), full((256, 128)), full((1, 128)),
            full((1, 128)), full((1, 1)),
        ],
        out_specs=pl.BlockSpec((_BB, 1), lambda i: (i, 0)),
        out_shape=jax.ShapeDtypeStruct((_B, 1), jnp.float32),
        compiler_params=pltpu.CompilerParams(
            dimension_semantics=("arbitrary",),
        ),
    )(uc, pc, *idx2, *tabs_t, *w1_parts,
      b1.reshape(1, 256), W2, b2.reshape(1, 128), Wfc.reshape(1, 128),
      bfc.reshape(1, 1))


def kernel(users, products, prices, skintypes, skintones, concerns, brands,
           user_emb, product_emb, price_emb, skintype_emb, skintone_emb,
           concern_emb, brand_emb, W1, b1, W2, b2, Wfc, bfc):
    users = users.astype(jnp.int32)
    products = products.astype(jnp.int32)
    ut128 = _pack_transposed(user_emb.T, 1000000)
    pt128 = _pack_transposed(product_emb.T, 100000)
    uidx = jnp.where(users < _H_U, users, users - _H_U) >> 1
    pidx = jnp.where(products < _H_P, products, products - _H_P) >> 1
    uc = _sc_gather(uidx, ut128)
    pc = _sc_gather(pidx, pt128)
    idx2 = tuple(x.astype(jnp.int32).reshape(_B, 1) for x in
                 (users, products, prices, skintypes, skintones, concerns,
                  brands))
    tabs_t = (price_emb.T, skintype_emb.T, skintone_emb.T, concern_emb.T,
              brand_emb.T)
    return _mlp(uc, pc, idx2, tabs_t, W1, b1, W2, b2, Wfc, bfc)


# speedup vs baseline: 1.2022x; 1.0002x over previous
import functools

import jax
import jax.numpy as jnp
from jax import lax
from jax.experimental import pallas as pl
from jax.experimental.pallas import tpu as pltpu
from jax.experimental.pallas import tpu_sc as plsc

_B = 16384
_NC = 2
_NS = 16
_NW = _NC * _NS
_BPW = _B // _NW
_HB = _BPW // 2


_CB = 24576


def _packed_half(n_rows):
    return ((n_rows // 2 + _CB - 1) // _CB) * _CB


_H_U = _packed_half(1000000)
_H_P = _packed_half(100000)


def _tr_body(xlo_ref, xhi_ref, eye_ref, o_ref):
    del eye_ref
    xt_lo = pltpu.bitcast(xlo_ref[...].T.astype(jnp.bfloat16), jnp.float32)
    xt_hi = pltpu.bitcast(xhi_ref[...].T.astype(jnp.bfloat16), jnp.float32)
    o_ref[...] = jnp.concatenate([xt_lo, xt_hi], axis=1)


def _pack_transposed(tab_t, n_rows):
    nblk = (n_rows // 2 + _CB - 1) // _CB
    half = nblk * _CB
    last = (n_rows + _CB - 1) // _CB - 1
    eye = jnp.eye(64, dtype=jnp.float32)
    return pl.pallas_call(
        _tr_body,
        grid=(nblk,),
        in_specs=[
            pl.BlockSpec((64, _CB), lambda i: (0, i)),
            pl.BlockSpec(
                (64, _CB),
                lambda i, _n=nblk, _l=last: (0, jnp.minimum(i + _n, _l))),
            pl.BlockSpec((64, 64), lambda i: (0, 0)),
        ],
        out_specs=pl.BlockSpec((_CB // 2, 128), lambda i: (i, 0)),
        out_shape=jax.ShapeDtypeStruct((half // 2, 128), jnp.float32),
        compiler_params=pltpu.CompilerParams(
            dimension_semantics=("parallel",),
        ),
    )(tab_t, tab_t, eye)


_GC = 128
_NCH = _BPW // _GC


def _sc_gather_body(t_i, t_t, t_o, idx_v, buf_a, buf_b, sem_a, sem_b):
    wid = lax.axis_index("s") * _NC + lax.axis_index("c")
    base = wid * _BPW
    for j in range(_NCH):
        pltpu.sync_copy(t_i.at[pl.ds(base + _GC * j, _GC)], idx_v.at[j])
    bufs = (buf_a, buf_b)
    sems = (sem_a, sem_b)

    def fire(j):
        return pltpu.async_copy(t_t.at[idx_v.at[j]], bufs[j % 2], sems[j % 2])

    def drain(j, d):
        d.wait()
        pltpu.sync_copy(bufs[j % 2], t_o.at[pl.ds(base + _GC * j, _GC)])

    prev = fire(0)
    for j in range(1, _NCH):
        cur = fire(j)
        drain(j - 1, prev)
        prev = cur
    drain(_NCH - 1, prev)


_sc_gather = functools.partial(
    pl.kernel,
    out_type=jax.ShapeDtypeStruct((_B, 128), jnp.float32),
    mesh=plsc.VectorSubcoreMesh(core_axis_name="c", subcore_axis_name="s"),
    scratch_types=[
        pltpu.VMEM((_NCH, _GC), jnp.int32),
        pltpu.VMEM((_GC, 128), jnp.float32),
        pltpu.VMEM((_GC, 128), jnp.float32),
        pltpu.SemaphoreType.DMA,
        pltpu.SemaphoreType.DMA,
    ],
)(_sc_gather_body)


_BB = 2048


def _extract_row(chunk, idx, half):
    bits = pltpu.bitcast(chunk, jnp.int32)
    h = idx >= half
    sel = jnp.where(h, bits[:, 64:128], bits[:, 0:64])
    q = (idx % 2) == 1
    word = jnp.where(q, sel & jnp.int32(-65536), sel << 16)
    return pltpu.bitcast(word, jnp.float32).astype(jnp.bfloat16)


def _onehot_contrib(idx, tab_t, w1k, n):
    bf = jnp.bfloat16
    oh = (idx == lax.broadcasted_iota(jnp.int32, (1, n), 1)).astype(bf)
    t1 = lax.dot_general(oh, tab_t[...], (((1,), (1,)), ((), ())),
                         preferred_element_type=jnp.float32)
    return jnp.dot(t1.astype(bf), w1k[...],
                   preferred_element_type=jnp.float32)


def _mlp_body(uc, pc, ui, pi, pri, sti, ski, coi, bri,
              prt, stt, skt, cot, brt,
              w1u, w1p, w1pr, w1st, w1sk, w1co, w1br,
              b1, w2, b2, wfc_t, bfc, o):
    f32 = jnp.float32
    bf = jnp.bfloat16
    u = _extract_row(uc[...], ui[...], _H_U)
    p = _extract_row(pc[...], pi[...], _H_P)
    acc = jnp.dot(u, w1u[...], preferred_element_type=f32)
    acc += jnp.dot(p, w1p[...], preferred_element_type=f32)
    acc += _onehot_contrib(pri[...], prt, w1pr, 100)
    acc += _onehot_contrib(sti[...], stt, w1st, 8)
    acc += _onehot_contrib(ski[...], skt, w1sk, 16)
    acc += _onehot_contrib(coi[...], cot, w1co, 32)
    acc += _onehot_contrib(bri[...], brt, w1br, 1000)
    h1 = jnp.maximum(acc + b1[...], 0.0)
    h2 = jnp.maximum(
        jnp.dot(h1.astype(bf), w2[...],
                preferred_element_type=f32) + b2[...], 0.0)
    logit = jnp.sum(h2 * wfc_t[...], axis=1, keepdims=True) + bfc[...]
    o[...] = jax.nn.sigmoid(logit)


def _mlp(uc, pc, idx2, tabs_t, W1, b1, W2, b2, Wfc, bfc):
    bf = jnp.bfloat16
    W1 = W1.astype(bf)
    W2 = W2.astype(bf)
    w1_parts = (W1[0:64], W1[64:128], W1[128:160], W1[160:192],
                W1[192:224], W1[224:256], W1[256:288])
    tabs_t = tuple(t.astype(bf) for t in tabs_t)
    grid = (_B // _BB,)
    chunk_spec = pl.BlockSpec((_BB, 128), lambda i: (i, 0))
    idx_spec = pl.BlockSpec((_BB, 1), lambda i: (i, 0))
    full = lambda shape: pl.BlockSpec(shape, lambda i: (0, 0))
    return pl.pallas_call(
        _mlp_body,
        grid=grid,
        in_specs=[
            chunk_spec, chunk_spec,
            idx_spec, idx_spec, idx_spec, idx_spec, idx_spec, idx_spec,
            idx_spec,
            full((32, 100)), full((32, 8)), full((32, 16)), full((32, 32)),
            full((32, 1000)),
            full((64, 256)), full((64, 256)), full((32, 256)), full((32, 256)),
            full((32, 256)), full((32, 256)), full((32, 256)),
            full((1, 256)), full((256, 128)), full((1, 128)),
            full((1, 128)), full((1, 1)),
        ],
        out_specs=pl.BlockSpec((_BB, 1), lambda i: (i, 0)),
        out_shape=jax.ShapeDtypeStruct((_B, 1), jnp.float32),
        compiler_params=pltpu.CompilerParams(
            dimension_semantics=("arbitrary",),
        ),
    )(uc, pc, *idx2, *tabs_t, *w1_parts,
      b1.reshape(1, 256), W2, b2.reshape(1, 128), Wfc.reshape(1, 128),
      bfc.reshape(1, 1))


def kernel(users, products, prices, skintypes, skintones, concerns, brands,
           user_emb, product_emb, price_emb, skintype_emb, skintone_emb,
           concern_emb, brand_emb, W1, b1, W2, b2, Wfc, bfc):
    users = users.astype(jnp.int32)
    products = products.astype(jnp.int32)
    ut128 = _pack_transposed(user_emb.T, 1000000)
    pt128 = _pack_transposed(product_emb.T, 100000)
    uidx = jnp.where(users < _H_U, users, users - _H_U) >> 1
    pidx = jnp.where(products < _H_P, products, products - _H_P) >> 1
    uc = _sc_gather(uidx, ut128)
    pc = _sc_gather(pidx, pt128)
    idx2 = tuple(x.astype(jnp.int32).reshape(_B, 1) for x in
                 (users, products, prices, skintypes, skintones, concerns,
                  brands))
    tabs_t = (price_emb.T, skintype_emb.T, skintone_emb.T, concern_emb.T,
              brand_emb.T)
    return _mlp(uc, pc, idx2, tabs_t, W1, b1, W2, b2, Wfc, bfc)
